# baseline (speedup 1.0000x reference)
import functools

import jax
import jax.numpy as jnp
from jax import lax
from jax.experimental import pallas as pl
from jax.experimental.pallas import tpu as pltpu
from jax.experimental.pallas import tpu_sc as plsc

VOCAB = 100000
SHARD = 50000
TAB_PAD = SHARD + 8
D = 128
B = 16384 * 50
NC, NS = 2, 16
NW = NC * NS
IDX_COLS = 128
IDX_ROWS = B // IDX_COLS
IDX_ROWS_PER_W = IDX_ROWS // NW

_mesh = plsc.VectorSubcoreMesh(core_axis_name="c", subcore_axis_name="s")


@functools.partial(
    pl.kernel,
    mesh=_mesh,
    out_type=jax.ShapeDtypeStruct((IDX_ROWS, IDX_COLS), jnp.int32),
    scratch_types=[
        pltpu.VMEM((IDX_ROWS_PER_W, IDX_COLS), jnp.int32),
    ],
)
def _sc_clamp(idx_hbm, out_hbm, ibuf):
    wid = lax.axis_index("s") * NC + lax.axis_index("c")
    base = wid * IDX_ROWS_PER_W
    pltpu.sync_copy(idx_hbm.at[pl.ds(base, IDX_ROWS_PER_W)], ibuf)

    def row(r, _):
        def col(c, _):
            sl = pl.ds(c * 16, 16)
            ibuf[r, sl] = jnp.minimum(ibuf[r, sl], SHARD)
            return 0
        lax.fori_loop(0, IDX_COLS // 16, col, 0)
        return 0

    lax.fori_loop(0, IDX_ROWS_PER_W, row, 0)
    pltpu.sync_copy(ibuf, out_hbm.at[pl.ds(base, IDX_ROWS_PER_W)])


R = 2048
IR = R // IDX_COLS
GRID = B // R


def _tc_body(idx_ref, tab_hbm, out_ref, tab_v, sem):
    @pl.when(pl.program_id(0) == 0)
    def _():
        cp = pltpu.make_async_copy(tab_hbm, tab_v.at[pl.ds(0, SHARD)], sem)
        cp.start()
        tab_v[pl.ds(SHARD, 8), :] = jnp.zeros((8, D), jnp.float32)
        cp.wait()

    def step(i, _):
        base = i * 16
        for h in range(2):
            rows = [tab_v[pl.ds(idx_ref[base + h * 8 + k], 1), :]
                    for k in range(8)]
            out_ref[pl.ds(base + h * 8, 8), :] = jnp.concatenate(rows, axis=0)
        return 0

    lax.fori_loop(0, R // 16, step, 0)


def _tc_gather(idx2d, tab):
    return pl.pallas_call(
        _tc_body,
        grid=(GRID,),
        in_specs=[
            pl.BlockSpec((R,), lambda g: (g,),
                         memory_space=pltpu.SMEM),
            pl.BlockSpec(memory_space=pltpu.MemorySpace.HBM),
        ],
        out_specs=pl.BlockSpec((R, D), lambda g: (g, 0)),
        out_shape=jax.ShapeDtypeStruct((B, D), jnp.float32),
        scratch_shapes=[
            pltpu.VMEM((TAB_PAD, D), jnp.float32),
            pltpu.SemaphoreType.DMA,
        ],
        compiler_params=pltpu.CompilerParams(
            dimension_semantics=("arbitrary",)),
    )(idx2d, tab)


def kernel(input, weight):
    idx = input.astype(jnp.int32).reshape(IDX_ROWS, IDX_COLS)
    idx_clamped = _sc_clamp(idx)
    out = _tc_gather(idx_clamped.reshape(B), weight)
    return out.reshape(input.shape[0], input.shape[1], D)


# speedup vs baseline: 1.0336x; 1.0336x over previous
import functools

import jax
import jax.numpy as jnp
from jax import lax
from jax.experimental import pallas as pl
from jax.experimental.pallas import tpu as pltpu
from jax.experimental.pallas import tpu_sc as plsc

VOCAB = 100000
SHARD = 50000
TAB_PAD = SHARD + 8
D = 128
B = 16384 * 50
NC, NS = 2, 16
NW = NC * NS
IDX_COLS = 128
IDX_ROWS = B // IDX_COLS
IDX_ROWS_PER_W = IDX_ROWS // NW

_mesh = plsc.VectorSubcoreMesh(core_axis_name="c", subcore_axis_name="s")


@functools.partial(
    pl.kernel,
    mesh=_mesh,
    out_type=jax.ShapeDtypeStruct((IDX_ROWS, IDX_COLS), jnp.int32),
    scratch_types=[
        pltpu.VMEM((IDX_ROWS_PER_W, IDX_COLS), jnp.int32),
    ],
)
def _sc_clamp(idx_hbm, out_hbm, ibuf):
    wid = lax.axis_index("s") * NC + lax.axis_index("c")
    base = wid * IDX_ROWS_PER_W
    pltpu.sync_copy(idx_hbm.at[pl.ds(base, IDX_ROWS_PER_W)], ibuf)

    def row(r, _):
        def col(c, _):
            sl = pl.ds(c * 16, 16)
            ibuf[r, sl] = jnp.minimum(ibuf[r, sl], SHARD)
            return 0
        lax.fori_loop(0, IDX_COLS // 16, col, 0)
        return 0

    lax.fori_loop(0, IDX_ROWS_PER_W, row, 0)
    pltpu.sync_copy(ibuf, out_hbm.at[pl.ds(base, IDX_ROWS_PER_W)])


R = 4096
IR = R // IDX_COLS
GRID = B // R


def _tc_body(idx_ref, tab_hbm, out_ref, tab_v, sem):
    @pl.when(pl.program_id(0) == 0)
    def _():
        cp = pltpu.make_async_copy(tab_hbm, tab_v.at[pl.ds(0, SHARD)], sem)
        cp.start()
        tab_v[pl.ds(SHARD, 8), :] = jnp.zeros((8, D), jnp.float32)
        cp.wait()

    def step(i, _):
        base = i * 16
        for k in range(16):
            j = idx_ref[base + k]
            out_ref[pl.ds(base + k, 1), :] = tab_v[pl.ds(j, 1), :]
        return 0

    lax.fori_loop(0, R // 16, step, 0)


def _tc_gather(idx2d, tab):
    return pl.pallas_call(
        _tc_body,
        grid=(GRID,),
        in_specs=[
            pl.BlockSpec((R,), lambda g: (g,),
                         memory_space=pltpu.SMEM),
            pl.BlockSpec(memory_space=pltpu.MemorySpace.HBM),
        ],
        out_specs=pl.BlockSpec((R, D), lambda g: (g, 0)),
        out_shape=jax.ShapeDtypeStruct((B, D), jnp.float32),
        scratch_shapes=[
            pltpu.VMEM((TAB_PAD, D), jnp.float32),
            pltpu.SemaphoreType.DMA,
        ],
        compiler_params=pltpu.CompilerParams(
            dimension_semantics=("arbitrary",)),
    )(idx2d, tab)


def kernel(input, weight):
    idx = input.astype(jnp.int32).reshape(IDX_ROWS, IDX_COLS)
    idx_clamped = _sc_clamp(idx)
    out = _tc_gather(idx_clamped.reshape(B), weight)
    return out.reshape(input.shape[0], input.shape[1], D)


# speedup vs baseline: 1.0342x; 1.0006x over previous
import functools

import jax
import jax.numpy as jnp
from jax import lax
from jax.experimental import pallas as pl
from jax.experimental.pallas import tpu as pltpu
from jax.experimental.pallas import tpu_sc as plsc

VOCAB = 100000
SHARD = 50000
TAB_PAD = SHARD + 8
D = 128
B = 16384 * 50
NC, NS = 2, 16
NW = NC * NS
IDX_COLS = 128
IDX_ROWS = B // IDX_COLS
IDX_ROWS_PER_W = IDX_ROWS // NW

_mesh = plsc.VectorSubcoreMesh(core_axis_name="c", subcore_axis_name="s")


@functools.partial(
    pl.kernel,
    mesh=_mesh,
    out_type=jax.ShapeDtypeStruct((IDX_ROWS, IDX_COLS), jnp.int32),
    scratch_types=[
        pltpu.VMEM((IDX_ROWS_PER_W, IDX_COLS), jnp.int32),
    ],
)
def _sc_clamp(idx_hbm, out_hbm, ibuf):
    wid = lax.axis_index("s") * NC + lax.axis_index("c")
    base = wid * IDX_ROWS_PER_W
    pltpu.sync_copy(idx_hbm.at[pl.ds(base, IDX_ROWS_PER_W)], ibuf)

    def row(r, _):
        def col(c, _):
            sl = pl.ds(c * 16, 16)
            ibuf[r, sl] = jnp.minimum(ibuf[r, sl], SHARD)
            return 0
        lax.fori_loop(0, IDX_COLS // 16, col, 0)
        return 0

    lax.fori_loop(0, IDX_ROWS_PER_W, row, 0)
    pltpu.sync_copy(ibuf, out_hbm.at[pl.ds(base, IDX_ROWS_PER_W)])


R = 8192
IR = R // IDX_COLS
GRID = B // R


def _tc_body(idx_ref, tab_hbm, out_ref, tab_v, sem):
    @pl.when(pl.program_id(0) == 0)
    def _():
        cp = pltpu.make_async_copy(tab_hbm, tab_v.at[pl.ds(0, SHARD)], sem)
        cp.start()
        tab_v[pl.ds(SHARD, 8), :] = jnp.zeros((8, D), jnp.float32)
        cp.wait()

    def step(i, _):
        base = i * 16
        for k in range(16):
            j = idx_ref[base + k]
            out_ref[pl.ds(base + k, 1), :] = tab_v[pl.ds(j, 1), :]
        return 0

    lax.fori_loop(0, R // 16, step, 0)


def _tc_gather(idx2d, tab):
    return pl.pallas_call(
        _tc_body,
        grid=(GRID,),
        in_specs=[
            pl.BlockSpec((R,), lambda g: (g,),
                         memory_space=pltpu.SMEM),
            pl.BlockSpec(memory_space=pltpu.MemorySpace.HBM),
        ],
        out_specs=pl.BlockSpec((R, D), lambda g: (g, 0)),
        out_shape=jax.ShapeDtypeStruct((B, D), jnp.float32),
        scratch_shapes=[
            pltpu.VMEM((TAB_PAD, D), jnp.float32),
            pltpu.SemaphoreType.DMA,
        ],
        compiler_params=pltpu.CompilerParams(
            dimension_semantics=("arbitrary",)),
    )(idx2d, tab)


def kernel(input, weight):
    idx = input.astype(jnp.int32).reshape(IDX_ROWS, IDX_COLS)
    idx_clamped = _sc_clamp(idx)
    out = _tc_gather(idx_clamped.reshape(B), weight)
    return out.reshape(input.shape[0], input.shape[1], D)
